# baseline (speedup 1.0000x reference)
import jax
import jax.numpy as jnp
from jax.experimental import pallas as pl
from jax.experimental.pallas import tpu as pltpu
from jax.experimental.pallas import tpu_sc as plsc

_WINDOW = 128
_LANES = 128


def kernel(input_ids, table):
    batch, hist = input_ids.shape
    vocab, dim = table.shape
    num_indices = batch * hist
    assert num_indices % _WINDOW == 0

    mesh = plsc.VectorSubcoreMesh(core_axis_name="c", subcore_axis_name="s")

    @jax.jit
    def run(table, idx):
        padded = jnp.pad(table, ((0, 0), (0, _LANES - dim)))

        @pl.kernel(
            out_type=jax.ShapeDtypeStruct((num_indices, _LANES), table.dtype),
            mesh=mesh,
        )
        def gather_kernel(table_hbm, idx_hbm, out_hbm):
            def body(idx_vmem, out_vmem):
                pltpu.sync_copy(table_hbm.at[idx_vmem.at[0]], out_vmem)

            pltpu.emit_pipeline(
                body,
                grid=(num_indices // _WINDOW,),
                in_specs=[pl.BlockSpec((1, _WINDOW), lambda i: (0, i))],
                out_specs=[pl.BlockSpec((_WINDOW, _LANES), lambda i: (i, 0))],
                core_axis_name=("c", "s"),
                dimension_semantics=(pltpu.PARALLEL,),
            )(idx_hbm, out_hbm)

        out = gather_kernel(padded, idx)
        return out[:, :dim].reshape(batch, hist, dim)

    return run(table, input_ids.reshape(1, num_indices))
